# baseline (speedup 1.0000x reference)
import functools

import jax
import jax.numpy as jnp
from jax import lax
from jax.experimental import pallas as pl
from jax.experimental.pallas import tpu as pltpu
from jax.experimental.pallas import tpu_sc as plsc

N = 10000
NP = 10240
E = 320000
LANES = 16
NC = 2
NS = 16
NW = NC * NS
EPW = E // NW
ROWS_PER_TILE = NP // NS


def _sel_matrix(rows, cols, group):
    r = lax.broadcasted_iota(jnp.int32, (rows, cols), 0)
    c = lax.broadcasted_iota(jnp.int32, (rows, cols), 1)
    return jnp.where(r // group == c, 1.0, 0.0).astype(jnp.float32)


def _tc1_body(x_ref, w_ref, asrc_ref, adst_ref, hext_ref, oad_ref):
    h = jnp.dot(x_ref[...], w_ref[...], preferred_element_type=jnp.float32)
    hc = h.shape[1]
    s = _sel_matrix(hc, LANES, 16)
    hext_ref[:, :hc] = h
    hext_ref[:, hc:] = jnp.dot(h * asrc_ref[...], s,
                               preferred_element_type=jnp.float32)
    oad_ref[...] = jnp.dot(h * adst_ref[...], s,
                           preferred_element_type=jnp.float32)


def _tc1(x, w1, asrc_flat, adst_flat):
    hc = w1.shape[1]
    return pl.pallas_call(
        _tc1_body,
        out_shape=[
            jax.ShapeDtypeStruct((NP, hc + LANES), jnp.float32),
            jax.ShapeDtypeStruct((NP, LANES), jnp.float32),
        ],
    )(x, w1, asrc_flat, adst_flat)


def _tc2_body(accp_ref, b1_ref, w2_ref, as2_ref, ad2_ref, hext_ref, oad_ref):
    hc = accp_ref.shape[1] - LANES
    acc = accp_ref[:NP, :hc] + accp_ref[NP:, :hc]
    den = accp_ref[:NP, hc:] + accp_ref[NP:, hc:]
    recip = 1.0 / (den + 1e-16)
    r = lax.broadcasted_iota(jnp.int32, (LANES, hc), 0)
    c = lax.broadcasted_iota(jnp.int32, (LANES, hc), 1)
    expand = jnp.where(c // 16 == r, 1.0, 0.0).astype(jnp.float32)
    x2 = acc * jnp.dot(recip, expand, preferred_element_type=jnp.float32)
    x2 = x2 + b1_ref[...]
    x2 = jnp.where(x2 > 0.0, x2, jnp.exp(x2) - 1.0)
    h2 = jnp.dot(x2, w2_ref[...], preferred_element_type=jnp.float32)
    out = h2.shape[1]
    s2 = _sel_matrix(out, LANES, out)
    hext_ref[:, :out] = h2
    hext_ref[:, out:] = jnp.dot(h2 * as2_ref[...], s2,
                                preferred_element_type=jnp.float32)
    oad_ref[...] = jnp.dot(h2 * ad2_ref[...], s2,
                           preferred_element_type=jnp.float32)


def _tc2(accp, b1_flat, w2, as2_flat, ad2_flat):
    out = w2.shape[1]
    return pl.pallas_call(
        _tc2_body,
        out_shape=[
            jax.ShapeDtypeStruct((NP, out + LANES), jnp.float32),
            jax.ShapeDtypeStruct((NP, LANES), jnp.float32),
        ],
    )(accp, b1_flat, w2, as2_flat, ad2_flat)


def _tc3_body(accp_ref, b2_ref, out_ref):
    out = accp_ref.shape[1] - LANES
    acc = accp_ref[:NP, :out] + accp_ref[NP:, :out]
    den = accp_ref[:NP, out:] + accp_ref[NP:, out:]
    z = acc * (1.0 / (den[:, 0:1] + 1e-16)) + b2_ref[...]
    m = jnp.max(z, axis=1, keepdims=True)
    zs = z - m
    out_ref[...] = zs - jnp.log(jnp.sum(jnp.exp(zs), axis=1, keepdims=True))


def _tc3(accp, b2_flat):
    out = accp.shape[1] - LANES
    return pl.pallas_call(
        _tc3_body,
        out_shape=jax.ShapeDtypeStruct((NP, out), jnp.float32),
    )(accp, b2_flat)


def _make_edge_agg(d, heads, BLK):
    NBLK = EPW // BLK
    nseg = d // LANES
    dext = d + LANES
    hos = [g * heads // nseg for g in range(nseg)]
    mesh = plsc.VectorSubcoreMesh(core_axis_name="c", subcore_axis_name="s",
                                  num_cores=NC, num_subcores=NS)

    @functools.partial(
        pl.kernel,
        mesh=mesh,
        compiler_params=pltpu.CompilerParams(use_tc_tiling_on_sc=False),
        out_type=jax.ShapeDtypeStruct((2 * NP, dext), jnp.float32),
        scratch_types=[
            pltpu.VMEM((4, BLK), jnp.int32),
            pltpu.VMEM((4, BLK), jnp.int32),
            pltpu.VMEM((2, BLK, LANES), jnp.float32),
            pltpu.VMEM((2, BLK, dext), jnp.float32),
            pltpu.VMEM((2, BLK, dext), jnp.float32),
            pltpu.VMEM_SHARED((NP, dext), jnp.float32),
            pltpu.SemaphoreType.DMA((4,)),
            pltpu.SemaphoreType.DMA((2,)),
            pltpu.SemaphoreType.DMA((2,)),
        ],
    )
    def edge_kernel(src_hbm, dst_hbm, hext_hbm, ad_hbm, acc_out,
                    srci, dsti, adv, hvg, msg, acc_sh, isem, gsem, ssem):
        c = lax.axis_index("c")
        s = lax.axis_index("s")
        wid = c * NS + s

        zv = jnp.zeros((LANES,), jnp.float32)

        def zero_row(i, _):
            for g in range(nseg + 1):
                msg[0, i, pl.ds(g * LANES, LANES)] = zv
            return 0
        lax.fori_loop(0, BLK, zero_row, 0)
        r0 = s * ROWS_PER_TILE
        nch = ROWS_PER_TILE // BLK
        for j in range(nch):
            pltpu.async_copy(msg.at[0], acc_sh.at[pl.ds(r0 + j * BLK, BLK)],
                             gsem.at[0])
        for j in range(nch):
            pltpu.make_async_copy(msg.at[0],
                                  acc_sh.at[pl.ds(r0 + j * BLK, BLK)],
                                  gsem.at[0]).wait()
        plsc.subcore_barrier()

        ebase = wid * EPW

        def issue_idx(b, j):
            e0 = ebase + b * BLK
            pltpu.async_copy(src_hbm.at[pl.ds(e0, BLK)], srci.at[j], isem.at[j])
            pltpu.async_copy(dst_hbm.at[pl.ds(e0, BLK)], dsti.at[j], isem.at[j])

        def wait_idx(j):
            pltpu.make_async_copy(src_hbm.at[pl.ds(0, BLK)], srci.at[j],
                                  isem.at[j]).wait()
            pltpu.make_async_copy(dst_hbm.at[pl.ds(0, BLK)], dsti.at[j],
                                  isem.at[j]).wait()

        def issue_gathers(j, p):
            pltpu.async_copy(hext_hbm.at[srci.at[j]], hvg.at[p], gsem.at[p])
            pltpu.async_copy(ad_hbm.at[dsti.at[j]], adv.at[p], gsem.at[p])

        def wait_gathers(j, p):
            pltpu.make_async_copy(hext_hbm.at[srci.at[j]], hvg.at[p],
                                  gsem.at[p]).wait()
            pltpu.make_async_copy(ad_hbm.at[dsti.at[j]], adv.at[p],
                                  gsem.at[p]).wait()

        def drain_scatter(p):
            pltpu.make_async_copy(msg.at[p], acc_sh.at[dsti.at[0]],
                                  ssem.at[p]).wait()

        issue_idx(0, 0)
        issue_idx(1, 1)
        wait_idx(0)
        issue_gathers(0, 0)

        def blk_body(b, _):
            p = lax.rem(b, 2)
            q = lax.rem(b + 1, 2)

            @pl.when(b >= 2)
            def _():
                drain_scatter(p)

            @pl.when(b + 2 < NBLK)
            def _():
                issue_idx(b + 2, lax.rem(b + 2, 4))

            @pl.when(b + 1 < NBLK)
            def _():
                j1 = lax.rem(b + 1, 4)
                wait_idx(j1)
                issue_gathers(j1, q)

            wait_gathers(lax.rem(b, 4), p)

            @plsc.parallel_loop(0, BLK, unroll=16)
            def _(e):
                a = hvg[p, e, pl.ds(d, LANES)] + adv[p, e, :]
                a = jnp.where(a > 0.0, a, 0.2 * a)
                w = jnp.exp(a)
                msg[p, e, pl.ds(d, LANES)] = w
                for g in range(nseg):
                    seg = hvg[p, e, pl.ds(g * LANES, LANES)]
                    msg[p, e, pl.ds(g * LANES, LANES)] = seg * w[hos[g]]

            pltpu.async_copy(msg.at[p], acc_sh.at[dsti.at[lax.rem(b, 4)]],
                             ssem.at[p], add=True)
            return 0
        lax.fori_loop(0, NBLK, blk_body, 0)

        drain_scatter(0)
        drain_scatter(1)

        plsc.subcore_barrier()

        ro = c * NP + s * ROWS_PER_TILE
        pltpu.async_copy(acc_sh.at[pl.ds(r0, BLK)], msg.at[0], gsem.at[0])
        for j in range(nch):
            p = j % 2
            q = (j + 1) % 2
            if j + 1 < nch:
                if j >= 1:
                    pltpu.make_async_copy(
                        msg.at[q],
                        acc_out.at[pl.ds(ro + (j - 1) * BLK, BLK)],
                        ssem.at[q]).wait()
                pltpu.async_copy(acc_sh.at[pl.ds(r0 + (j + 1) * BLK, BLK)],
                                 msg.at[q], gsem.at[q])
            pltpu.make_async_copy(acc_sh.at[pl.ds(r0 + j * BLK, BLK)],
                                  msg.at[p], gsem.at[p]).wait()
            pltpu.async_copy(msg.at[p], acc_out.at[pl.ds(ro + j * BLK, BLK)],
                             ssem.at[p])
        for j in (nch - 2, nch - 1):
            pltpu.make_async_copy(msg.at[j % 2],
                                  acc_out.at[pl.ds(ro + j * BLK, BLK)],
                                  ssem.at[j % 2]).wait()

    return edge_kernel


def kernel(x, edge_index, W1, att_src1, att_dst1, b1, W2, att_src2, att_dst2, b2):
    src = edge_index[0]
    dst = edge_index[1]
    xp = jnp.pad(x, ((0, NP - N), (0, 0)))

    hext1, ad1 = _tc1(xp, W1, att_src1.reshape(1, -1), att_dst1.reshape(1, -1))
    edge1 = _make_edge_agg(W1.shape[1], att_src1.shape[1], 40)
    accp1 = edge1(src, dst, hext1, ad1)

    hext2, ad2 = _tc2(accp1, b1.reshape(1, -1), W2,
                      att_src2.reshape(1, -1), att_dst2.reshape(1, -1))
    edge2 = _make_edge_agg(W2.shape[1], att_src2.shape[1], 80)
    accp2 = edge2(src, dst, hext2, ad2)

    return _tc3(accp2, b2.reshape(1, -1))[:N]


# speedup vs baseline: 1.1119x; 1.1119x over previous
import functools

import jax
import jax.numpy as jnp
from jax import lax
from jax.experimental import pallas as pl
from jax.experimental.pallas import tpu as pltpu
from jax.experimental.pallas import tpu_sc as plsc

N = 10000
NP = 10240
E = 320000
LANES = 16
NC = 2
NS = 16
NW = NC * NS
EPW = E // NW
ROWS_PER_TILE = NP // NS


def _sel_matrix(rows, cols, group):
    r = lax.broadcasted_iota(jnp.int32, (rows, cols), 0)
    c = lax.broadcasted_iota(jnp.int32, (rows, cols), 1)
    return jnp.where(r // group == c, 1.0, 0.0).astype(jnp.float32)


def _tc1_body(x_ref, w_ref, asrc_ref, adst_ref, hext_ref, oad_ref):
    h = jnp.dot(x_ref[...], w_ref[...], preferred_element_type=jnp.float32)
    hc = h.shape[1]
    s = _sel_matrix(hc, LANES, 16)
    hext_ref[:, :hc] = h
    hext_ref[:, hc:] = jnp.dot(h * asrc_ref[...], s,
                               preferred_element_type=jnp.float32)
    oad_ref[...] = jnp.dot(h * adst_ref[...], s,
                           preferred_element_type=jnp.float32)


def _tc1(x, w1, asrc_flat, adst_flat):
    hc = w1.shape[1]
    return pl.pallas_call(
        _tc1_body,
        out_shape=[
            jax.ShapeDtypeStruct((NP, hc + LANES), jnp.float32),
            jax.ShapeDtypeStruct((NP, LANES), jnp.float32),
        ],
    )(x, w1, asrc_flat, adst_flat)


def _tc2_body(accp_ref, b1_ref, w2_ref, as2_ref, ad2_ref, hext_ref, oad_ref):
    hc = accp_ref.shape[1] - LANES
    acc = accp_ref[:NP, :hc] + accp_ref[NP:, :hc]
    den = accp_ref[:NP, hc:] + accp_ref[NP:, hc:]
    recip = 1.0 / (den + 1e-16)
    r = lax.broadcasted_iota(jnp.int32, (LANES, hc), 0)
    c = lax.broadcasted_iota(jnp.int32, (LANES, hc), 1)
    expand = jnp.where(c // 16 == r, 1.0, 0.0).astype(jnp.float32)
    x2 = acc * jnp.dot(recip, expand, preferred_element_type=jnp.float32)
    x2 = x2 + b1_ref[...]
    x2 = jnp.where(x2 > 0.0, x2, jnp.exp(x2) - 1.0)
    h2 = jnp.dot(x2, w2_ref[...], preferred_element_type=jnp.float32)
    out = h2.shape[1]
    s2 = _sel_matrix(out, LANES, out)
    hext_ref[:, :out] = h2
    hext_ref[:, out:] = jnp.dot(h2 * as2_ref[...], s2,
                                preferred_element_type=jnp.float32)
    oad_ref[...] = jnp.dot(h2 * ad2_ref[...], s2,
                           preferred_element_type=jnp.float32)


def _tc2(accp, b1_flat, w2, as2_flat, ad2_flat):
    out = w2.shape[1]
    return pl.pallas_call(
        _tc2_body,
        out_shape=[
            jax.ShapeDtypeStruct((NP, out + LANES), jnp.float32),
            jax.ShapeDtypeStruct((NP, LANES), jnp.float32),
        ],
    )(accp, b1_flat, w2, as2_flat, ad2_flat)


def _tc3_body(accp_ref, b2_ref, out_ref):
    out = accp_ref.shape[1] - LANES
    acc = accp_ref[:NP, :out] + accp_ref[NP:, :out]
    den = accp_ref[:NP, out:] + accp_ref[NP:, out:]
    z = acc * (1.0 / (den[:, 0:1] + 1e-16)) + b2_ref[...]
    m = jnp.max(z, axis=1, keepdims=True)
    zs = z - m
    out_ref[...] = zs - jnp.log(jnp.sum(jnp.exp(zs), axis=1, keepdims=True))


def _tc3(accp, b2_flat):
    out = accp.shape[1] - LANES
    return pl.pallas_call(
        _tc3_body,
        out_shape=jax.ShapeDtypeStruct((NP, out), jnp.float32),
    )(accp, b2_flat)


def _make_edge_agg(d, heads, BLK):
    NBLK = EPW // BLK
    nseg = d // LANES
    dext = d + LANES
    hos = [g * heads // nseg for g in range(nseg)]
    mesh = plsc.VectorSubcoreMesh(core_axis_name="c", subcore_axis_name="s",
                                  num_cores=NC, num_subcores=NS)

    @functools.partial(
        pl.kernel,
        mesh=mesh,
        compiler_params=pltpu.CompilerParams(use_tc_tiling_on_sc=False),
        out_type=jax.ShapeDtypeStruct((2 * NP, dext), jnp.float32),
        scratch_types=[
            pltpu.VMEM((4, BLK), jnp.int32),
            pltpu.VMEM((4, BLK), jnp.int32),
            pltpu.VMEM((2, BLK, LANES), jnp.float32),
            pltpu.VMEM((2, BLK, dext), jnp.float32),
            pltpu.VMEM((2, BLK, dext), jnp.float32),
            pltpu.VMEM_SHARED((NP, dext), jnp.float32),
            pltpu.SemaphoreType.DMA((4,)),
            pltpu.SemaphoreType.DMA((2,)),
            pltpu.SemaphoreType.DMA((2,)),
        ],
    )
    def edge_kernel(src_hbm, dst_hbm, hext_hbm, ad_hbm, acc_out,
                    srci, dsti, adv, hvg, msg, acc_sh, isem, gsem, ssem):
        c = lax.axis_index("c")
        s = lax.axis_index("s")
        wid = c * NS + s

        zv = jnp.zeros((LANES,), jnp.float32)

        def zero_row(i, _):
            for g in range(nseg + 1):
                msg[0, i, pl.ds(g * LANES, LANES)] = zv
            return 0
        lax.fori_loop(0, BLK, zero_row, 0)
        r0 = s * ROWS_PER_TILE
        nch = ROWS_PER_TILE // BLK
        for j in range(nch):
            pltpu.async_copy(msg.at[0], acc_sh.at[pl.ds(r0 + j * BLK, BLK)],
                             gsem.at[0])
        for j in range(nch):
            pltpu.make_async_copy(msg.at[0],
                                  acc_sh.at[pl.ds(r0 + j * BLK, BLK)],
                                  gsem.at[0]).wait()
        plsc.subcore_barrier()

        ebase = wid * EPW

        def issue_idx(b, j):
            e0 = ebase + b * BLK
            pltpu.async_copy(src_hbm.at[pl.ds(e0, BLK)], srci.at[j], isem.at[j])
            pltpu.async_copy(dst_hbm.at[pl.ds(e0, BLK)], dsti.at[j], isem.at[j])

        def wait_idx(j):
            pltpu.make_async_copy(src_hbm.at[pl.ds(0, BLK)], srci.at[j],
                                  isem.at[j]).wait()
            pltpu.make_async_copy(dst_hbm.at[pl.ds(0, BLK)], dsti.at[j],
                                  isem.at[j]).wait()

        def issue_gathers(j, p):
            pltpu.async_copy(hext_hbm.at[srci.at[j]], hvg.at[p], gsem.at[p])
            pltpu.async_copy(ad_hbm.at[dsti.at[j]], adv.at[p], gsem.at[p])

        def wait_gathers(j, p):
            pltpu.make_async_copy(hext_hbm.at[srci.at[j]], hvg.at[p],
                                  gsem.at[p]).wait()
            pltpu.make_async_copy(ad_hbm.at[dsti.at[j]], adv.at[p],
                                  gsem.at[p]).wait()

        def drain_scatter(p):
            pltpu.make_async_copy(msg.at[p], acc_sh.at[dsti.at[0]],
                                  ssem.at[p]).wait()

        issue_idx(0, 0)
        issue_idx(1, 1)
        wait_idx(0)
        issue_gathers(0, 0)

        def blk_body(b, _):
            p = lax.rem(b, 2)
            q = lax.rem(b + 1, 2)

            @pl.when(b >= 2)
            def _():
                drain_scatter(p)

            @pl.when(b + 2 < NBLK)
            def _():
                issue_idx(b + 2, lax.rem(b + 2, 4))

            @pl.when(b + 1 < NBLK)
            def _():
                j1 = lax.rem(b + 1, 4)
                wait_idx(j1)
                issue_gathers(j1, q)

            wait_gathers(lax.rem(b, 4), p)

            @plsc.parallel_loop(0, BLK, unroll=8)
            def _(e):
                a = hvg[p, e, pl.ds(d, LANES)] + adv[p, e, :]
                a = jnp.where(a > 0.0, a, 0.2 * a)
                w = jnp.exp(a)
                msg[p, e, pl.ds(d, LANES)] = w
                for g in range(nseg):
                    seg = hvg[p, e, pl.ds(g * LANES, LANES)]
                    msg[p, e, pl.ds(g * LANES, LANES)] = seg * w[hos[g]]

            pltpu.async_copy(msg.at[p], acc_sh.at[dsti.at[lax.rem(b, 4)]],
                             ssem.at[p], add=True)
            return 0
        lax.fori_loop(0, NBLK, blk_body, 0)

        drain_scatter(0)
        drain_scatter(1)

        plsc.subcore_barrier()

        ro = c * NP + s * ROWS_PER_TILE
        pltpu.async_copy(acc_sh.at[pl.ds(r0, BLK)], msg.at[0], gsem.at[0])
        for j in range(nch):
            p = j % 2
            q = (j + 1) % 2
            if j + 1 < nch:
                if j >= 1:
                    pltpu.make_async_copy(
                        msg.at[q],
                        acc_out.at[pl.ds(ro + (j - 1) * BLK, BLK)],
                        ssem.at[q]).wait()
                pltpu.async_copy(acc_sh.at[pl.ds(r0 + (j + 1) * BLK, BLK)],
                                 msg.at[q], gsem.at[q])
            pltpu.make_async_copy(acc_sh.at[pl.ds(r0 + j * BLK, BLK)],
                                  msg.at[p], gsem.at[p]).wait()
            pltpu.async_copy(msg.at[p], acc_out.at[pl.ds(ro + j * BLK, BLK)],
                             ssem.at[p])
        for j in (nch - 2, nch - 1):
            pltpu.make_async_copy(msg.at[j % 2],
                                  acc_out.at[pl.ds(ro + j * BLK, BLK)],
                                  ssem.at[j % 2]).wait()

    return edge_kernel


def kernel(x, edge_index, W1, att_src1, att_dst1, b1, W2, att_src2, att_dst2, b2):
    src = edge_index[0]
    dst = edge_index[1]
    xp = jnp.pad(x, ((0, NP - N), (0, 0)))

    hext1, ad1 = _tc1(xp, W1, att_src1.reshape(1, -1), att_dst1.reshape(1, -1))
    edge1 = _make_edge_agg(W1.shape[1], att_src1.shape[1], 40)
    accp1 = edge1(src, dst, hext1, ad1)

    hext2, ad2 = _tc2(accp1, b1.reshape(1, -1), W2,
                      att_src2.reshape(1, -1), att_dst2.reshape(1, -1))
    edge2 = _make_edge_agg(W2.shape[1], att_src2.shape[1], 80)
    accp2 = edge2(src, dst, hext2, ad2)

    return _tc3(accp2, b2.reshape(1, -1))[:N]


# speedup vs baseline: 1.1954x; 1.0751x over previous
import functools

import jax
import jax.numpy as jnp
from jax import lax
from jax.experimental import pallas as pl
from jax.experimental.pallas import tpu as pltpu
from jax.experimental.pallas import tpu_sc as plsc

N = 10000
NP = 10240
E = 320000
LANES = 16
NC = 2
NS = 16
NW = NC * NS
EPW = E // NW
ROWS_PER_TILE = NP // NS


def _sel_matrix(rows, cols, group):
    r = lax.broadcasted_iota(jnp.int32, (rows, cols), 0)
    c = lax.broadcasted_iota(jnp.int32, (rows, cols), 1)
    return jnp.where(r // group == c, 1.0, 0.0).astype(jnp.float32)


def _tc1_body(x_ref, w_ref, asrc_ref, adst_ref, hext_ref, oad_ref):
    h = jnp.dot(x_ref[...], w_ref[...], preferred_element_type=jnp.float32)
    hc = h.shape[1]
    s = _sel_matrix(hc, LANES, 16)
    hext_ref[:, :hc] = h
    hext_ref[:, hc:] = jnp.dot(h * asrc_ref[...], s,
                               preferred_element_type=jnp.float32)
    oad_ref[...] = jnp.dot(h * adst_ref[...], s,
                           preferred_element_type=jnp.float32)


def _tc1(x, w1, asrc_flat, adst_flat):
    hc = w1.shape[1]
    return pl.pallas_call(
        _tc1_body,
        out_shape=[
            jax.ShapeDtypeStruct((NP, hc + LANES), jnp.float32),
            jax.ShapeDtypeStruct((NP, LANES), jnp.float32),
        ],
    )(x, w1, asrc_flat, adst_flat)


def _tc2_body(accp_ref, b1_ref, w2_ref, as2_ref, ad2_ref, hext_ref, oad_ref):
    hc = accp_ref.shape[1] - LANES
    acc = accp_ref[:NP, :hc] + accp_ref[NP:, :hc]
    den = accp_ref[:NP, hc:] + accp_ref[NP:, hc:]
    recip = 1.0 / (den + 1e-16)
    r = lax.broadcasted_iota(jnp.int32, (LANES, hc), 0)
    c = lax.broadcasted_iota(jnp.int32, (LANES, hc), 1)
    expand = jnp.where(c // 16 == r, 1.0, 0.0).astype(jnp.float32)
    x2 = acc * jnp.dot(recip, expand, preferred_element_type=jnp.float32)
    x2 = x2 + b1_ref[...]
    x2 = jnp.where(x2 > 0.0, x2, jnp.exp(x2) - 1.0)
    h2 = jnp.dot(x2, w2_ref[...], preferred_element_type=jnp.float32)
    out = h2.shape[1]
    s2 = _sel_matrix(out, LANES, out)
    hext_ref[:, :out] = h2
    hext_ref[:, out:] = jnp.dot(h2 * as2_ref[...], s2,
                                preferred_element_type=jnp.float32)
    oad_ref[...] = jnp.dot(h2 * ad2_ref[...], s2,
                           preferred_element_type=jnp.float32)


def _tc2(accp, b1_flat, w2, as2_flat, ad2_flat):
    out = w2.shape[1]
    return pl.pallas_call(
        _tc2_body,
        out_shape=[
            jax.ShapeDtypeStruct((NP, out + LANES), jnp.float32),
            jax.ShapeDtypeStruct((NP, LANES), jnp.float32),
        ],
    )(accp, b1_flat, w2, as2_flat, ad2_flat)


def _tc3_body(accp_ref, b2_ref, out_ref):
    out = accp_ref.shape[1] - LANES
    acc = accp_ref[:NP, :out] + accp_ref[NP:, :out]
    den = accp_ref[:NP, out:] + accp_ref[NP:, out:]
    z = acc * (1.0 / (den[:, 0:1] + 1e-16)) + b2_ref[...]
    m = jnp.max(z, axis=1, keepdims=True)
    zs = z - m
    out_ref[...] = zs - jnp.log(jnp.sum(jnp.exp(zs), axis=1, keepdims=True))


def _tc3(accp, b2_flat):
    out = accp.shape[1] - LANES
    return pl.pallas_call(
        _tc3_body,
        out_shape=jax.ShapeDtypeStruct((NP, out), jnp.float32),
    )(accp, b2_flat)


def _make_edge_agg(d, heads, BLK, msg_dbl):
    NBLK = EPW // BLK
    nseg = d // LANES
    dext = d + LANES
    hos = [g * heads // nseg for g in range(nseg)]
    mesh = plsc.VectorSubcoreMesh(core_axis_name="c", subcore_axis_name="s",
                                  num_cores=NC, num_subcores=NS)

    @functools.partial(
        pl.kernel,
        mesh=mesh,
        compiler_params=pltpu.CompilerParams(use_tc_tiling_on_sc=False),
        out_type=jax.ShapeDtypeStruct((2 * NP, dext), jnp.float32),
        scratch_types=[
            pltpu.VMEM((4, BLK), jnp.int32),
            pltpu.VMEM((4, BLK), jnp.int32),
            pltpu.VMEM((2, BLK, LANES), jnp.float32),
            pltpu.VMEM((2, BLK, dext), jnp.float32),
            pltpu.VMEM((2 if msg_dbl else 1, BLK, dext), jnp.float32),
            pltpu.VMEM_SHARED((NP, dext), jnp.float32),
            pltpu.SemaphoreType.DMA((4,)),
            pltpu.SemaphoreType.DMA((2,)),
            pltpu.SemaphoreType.DMA((2,)),
        ],
    )
    def edge_kernel(src_hbm, dst_hbm, hext_hbm, ad_hbm, acc_out,
                    srci, dsti, adv, hvg, msg, acc_sh, isem, gsem, ssem):
        c = lax.axis_index("c")
        s = lax.axis_index("s")
        wid = c * NS + s

        zv = jnp.zeros((LANES,), jnp.float32)

        def zero_row(i, _):
            for g in range(nseg + 1):
                msg[0, i, pl.ds(g * LANES, LANES)] = zv
            return 0
        lax.fori_loop(0, BLK, zero_row, 0)
        r0 = s * ROWS_PER_TILE
        nch = ROWS_PER_TILE // BLK
        for j in range(nch):
            pltpu.async_copy(msg.at[0], acc_sh.at[pl.ds(r0 + j * BLK, BLK)],
                             gsem.at[0])
        for j in range(nch):
            pltpu.make_async_copy(msg.at[0],
                                  acc_sh.at[pl.ds(r0 + j * BLK, BLK)],
                                  gsem.at[0]).wait()
        plsc.subcore_barrier()

        ebase = wid * EPW

        def issue_idx(b, j):
            e0 = ebase + b * BLK
            pltpu.async_copy(src_hbm.at[pl.ds(e0, BLK)], srci.at[j], isem.at[j])
            pltpu.async_copy(dst_hbm.at[pl.ds(e0, BLK)], dsti.at[j], isem.at[j])

        def wait_idx(j):
            pltpu.make_async_copy(src_hbm.at[pl.ds(0, BLK)], srci.at[j],
                                  isem.at[j]).wait()
            pltpu.make_async_copy(dst_hbm.at[pl.ds(0, BLK)], dsti.at[j],
                                  isem.at[j]).wait()

        def issue_gathers(j, p):
            pltpu.async_copy(hext_hbm.at[srci.at[j]], hvg.at[p], gsem.at[p])
            pltpu.async_copy(ad_hbm.at[dsti.at[j]], adv.at[p], gsem.at[p])

        def wait_gathers(j, p):
            pltpu.make_async_copy(hext_hbm.at[srci.at[j]], hvg.at[p],
                                  gsem.at[p]).wait()
            pltpu.make_async_copy(ad_hbm.at[dsti.at[j]], adv.at[p],
                                  gsem.at[p]).wait()

        def drain_scatter(p):
            pltpu.make_async_copy(msg.at[p], acc_sh.at[dsti.at[0]],
                                  ssem.at[p]).wait()

        issue_idx(0, 0)
        issue_idx(1, 1)
        wait_idx(0)
        issue_gathers(0, 0)

        def blk_body(b, _):
            p = lax.rem(b, 2)
            q = lax.rem(b + 1, 2)
            mp = p if msg_dbl else 0

            if msg_dbl:
                @pl.when(b >= 2)
                def _():
                    drain_scatter(p)

            @pl.when(b + 2 < NBLK)
            def _():
                issue_idx(b + 2, lax.rem(b + 2, 4))

            @pl.when(b + 1 < NBLK)
            def _():
                j1 = lax.rem(b + 1, 4)
                wait_idx(j1)
                issue_gathers(j1, q)

            wait_gathers(lax.rem(b, 4), p)

            if not msg_dbl:
                @pl.when(b >= 1)
                def _():
                    drain_scatter(0)

            @plsc.parallel_loop(0, BLK, unroll=8)
            def _(e):
                a = hvg[p, e, pl.ds(d, LANES)] + adv[p, e, :]
                a = jnp.where(a > 0.0, a, 0.2 * a)
                w = jnp.exp(a)
                msg[mp, e, pl.ds(d, LANES)] = w
                for g in range(nseg):
                    seg = hvg[p, e, pl.ds(g * LANES, LANES)]
                    msg[mp, e, pl.ds(g * LANES, LANES)] = seg * w[hos[g]]

            pltpu.async_copy(msg.at[mp], acc_sh.at[dsti.at[lax.rem(b, 4)]],
                             ssem.at[mp], add=True)
            return 0
        lax.fori_loop(0, NBLK, blk_body, 0)

        drain_scatter(0)
        if msg_dbl:
            drain_scatter(1)

        plsc.subcore_barrier()

        ro = c * NP + s * ROWS_PER_TILE
        wob = [msg.at[0], msg.at[1] if msg_dbl else hvg.at[0]]
        pltpu.async_copy(acc_sh.at[pl.ds(r0, BLK)], wob[0], gsem.at[0])
        for j in range(nch):
            p = j % 2
            q = (j + 1) % 2
            if j + 1 < nch:
                if j >= 1:
                    pltpu.make_async_copy(
                        wob[q],
                        acc_out.at[pl.ds(ro + (j - 1) * BLK, BLK)],
                        ssem.at[q]).wait()
                pltpu.async_copy(acc_sh.at[pl.ds(r0 + (j + 1) * BLK, BLK)],
                                 wob[q], gsem.at[q])
            pltpu.make_async_copy(acc_sh.at[pl.ds(r0 + j * BLK, BLK)],
                                  wob[p], gsem.at[p]).wait()
            pltpu.async_copy(wob[p], acc_out.at[pl.ds(ro + j * BLK, BLK)],
                             ssem.at[p])
        for j in (nch - 2, nch - 1):
            pltpu.make_async_copy(wob[j % 2],
                                  acc_out.at[pl.ds(ro + j * BLK, BLK)],
                                  ssem.at[j % 2]).wait()

    return edge_kernel


def kernel(x, edge_index, W1, att_src1, att_dst1, b1, W2, att_src2, att_dst2, b2):
    src = edge_index[0]
    dst = edge_index[1]
    xp = jnp.pad(x, ((0, NP - N), (0, 0)))

    hext1, ad1 = _tc1(xp, W1, att_src1.reshape(1, -1), att_dst1.reshape(1, -1))
    edge1 = _make_edge_agg(W1.shape[1], att_src1.shape[1], 80, False)
    accp1 = edge1(src, dst, hext1, ad1)

    hext2, ad2 = _tc2(accp1, b1.reshape(1, -1), W2,
                      att_src2.reshape(1, -1), att_dst2.reshape(1, -1))
    edge2 = _make_edge_agg(W2.shape[1], att_src2.shape[1], 80, True)
    accp2 = edge2(src, dst, hext2, ad2)

    return _tc3(accp2, b2.reshape(1, -1))[:N]
